# baseline (speedup 1.0000x reference)
import functools

import jax
import jax.numpy as jnp
from jax import lax
from jax.experimental import pallas as pl
from jax.experimental.pallas import tpu as pltpu
from jax.experimental.pallas import tpu_sc as plsc

F32 = jnp.float32
NC, NS, LANES = 2, 16, 16
NW = NC * NS
CHUNK = 128
NBUF = 4
LAG = 2
_SC_PARAMS = pltpu.CompilerParams(use_tc_tiling_on_sc=False)


def _deg_call(n_pad, ch):
    mesh = plsc.VectorSubcoreMesh(core_axis_name="c", subcore_axis_name="s")
    rows_per_sub = n_pad // NS

    @functools.partial(
        pl.kernel, mesh=mesh, compiler_params=_SC_PARAMS,
        out_type=jax.ShapeDtypeStruct((NC, n_pad, LANES), F32),
        scratch_types=[
            pltpu.VMEM((ch, CHUNK), jnp.int32),
            pltpu.VMEM((CHUNK, LANES), F32),
            pltpu.VMEM_SHARED((n_pad, LANES), F32),
            pltpu.SemaphoreType.DMA,
        ],
    )
    def deg_kernel(dst_hbm, ones_hbm, zeros_hbm, out_hbm, dst_v, ones_v, acc_sh,
                   sem):
        c = lax.axis_index("c")
        s = lax.axis_index("s")
        wid = c * NS + s
        sl = pl.ds(s * rows_per_sub, rows_per_sub)
        pltpu.sync_copy(zeros_hbm, acc_sh.at[sl])
        pltpu.sync_copy(ones_hbm, ones_v)
        pltpu.sync_copy(dst_hbm.at[wid], dst_v)
        plsc.subcore_barrier()

        @pl.loop(0, ch)
        def _(j):
            pltpu.async_copy(ones_v, acc_sh.at[dst_v.at[j]], sem, add=True)

        @pl.loop(0, ch)
        def _(j):
            pltpu.make_async_copy(ones_v, acc_sh.at[dst_v.at[j]], sem).wait()

        plsc.subcore_barrier()
        pltpu.sync_copy(acc_sh.at[sl], out_hbm.at[c, sl])

    return deg_kernel


def _agg_call(n_pad, ch, d):
    mesh = plsc.VectorSubcoreMesh(core_axis_name="c", subcore_axis_name="s")
    rows_per_sub = n_pad // NS

    @functools.partial(
        pl.kernel, mesh=mesh, compiler_params=_SC_PARAMS,
        out_type=jax.ShapeDtypeStruct((NC, n_pad, d), F32),
        scratch_types=[
            pltpu.VMEM((ch, CHUNK), jnp.int32),
            pltpu.VMEM((ch, CHUNK), jnp.int32),
            pltpu.VMEM((NBUF, CHUNK, d), F32),
            pltpu.VMEM_SHARED((n_pad, d), F32),
            pltpu.VMEM_SHARED((n_pad, d), F32),
        ] + [pltpu.SemaphoreType.DMA] * (2 * NBUF),
    )
    def agg_kernel(hs_hbm, src_hbm, dst_hbm, zeros_hbm, out_hbm,
                   src_v, dst_v, rows_v, hs_sh, acc_sh, *sems):
        gsems, ssems = sems[:NBUF], sems[NBUF:]
        c = lax.axis_index("c")
        s = lax.axis_index("s")
        wid = c * NS + s
        sl = pl.ds(s * rows_per_sub, rows_per_sub)
        pltpu.sync_copy(zeros_hbm, acc_sh.at[sl])
        pltpu.sync_copy(hs_hbm.at[sl], hs_sh.at[sl])
        pltpu.sync_copy(src_hbm.at[wid], src_v)
        pltpu.sync_copy(dst_hbm.at[wid], dst_v)
        plsc.subcore_barrier()

        def gather(j, b):
            pltpu.async_copy(hs_sh.at[src_v.at[j]], rows_v.at[b], gsems[b])

        def wait_gather(j, b):
            pltpu.make_async_copy(hs_sh.at[src_v.at[j]], rows_v.at[b],
                                  gsems[b]).wait()

        def scatter(j, b):
            pltpu.async_copy(rows_v.at[b], acc_sh.at[dst_v.at[j]], ssems[b],
                             add=True)

        def wait_scatter(j, b):
            pltpu.make_async_copy(rows_v.at[b], acc_sh.at[dst_v.at[j]],
                                  ssems[b]).wait()

        for b in range(NBUF):
            gather(b, b)

        @pl.loop(0, ch // NBUF)
        def _(p):
            j0 = p * NBUF
            for b in range(NBUF):
                j = j0 + b
                wait_gather(j, b)
                scatter(j, b)
                bm = (b - LAG) % NBUF

                @pl.when((j >= LAG) & (j + NBUF - LAG < ch))
                def _():
                    wait_scatter(j - LAG, bm)
                    gather(j + NBUF - LAG, bm)

        for k in range(NBUF):
            jj = ch - NBUF + k
            wait_scatter(jj, jj % NBUF)

        plsc.subcore_barrier()
        pltpu.sync_copy(acc_sh.at[sl], out_hbm.at[c, sl])

    return agg_kernel


def _dis_from(deg_ref):
    deg = deg_ref[0, :, 0] + deg_ref[1, :, 0]
    return lax.rsqrt(jnp.maximum(deg, 1.0))


def _matmul_call(n_pad, d_in, h, blk):
    def body(x_ref, w_ref, o_ref):
        o_ref[...] = jnp.dot(x_ref[...], w_ref[...], preferred_element_type=F32)

    mblk = 2048
    return pl.pallas_call(
        body,
        grid=(n_pad // mblk,),
        in_specs=[
            pl.BlockSpec((mblk, d_in), lambda i: (i, 0)),
            pl.BlockSpec((d_in, h), lambda i: (0, 0)),
        ],
        out_specs=pl.BlockSpec((mblk, h), lambda i: (i, 0)),
        out_shape=jax.ShapeDtypeStruct((n_pad, h), F32),
    )


def _scale_call(n_pad, h, blk):
    def body(g_ref, deg_ref, o_ref):
        dis = _dis_from(deg_ref)
        o_ref[...] = g_ref[...] * dis[:, None]

    sblk = 2048
    return pl.pallas_call(
        body,
        grid=(n_pad // sblk,),
        in_specs=[
            pl.BlockSpec((sblk, h), lambda i: (i, 0)),
            pl.BlockSpec((NC, sblk, LANES), lambda i: (0, i, 0)),
        ],
        out_specs=pl.BlockSpec((sblk, h), lambda i: (i, 0)),
        out_shape=jax.ShapeDtypeStruct((n_pad, h), F32),
    )


def _tc2_call(n_pad, h, c_pad, blk):
    blk = 2048
    def body(acc_ref, hs1_ref, deg_ref, w2_ref, b1_ref, o_ref):
        dis = _dis_from(deg_ref)
        u = (acc_ref[0] + acc_ref[1] + hs1_ref[...]) * dis[:, None] + b1_ref[...]
        u = jnp.maximum(u, 0.0)
        o_ref[...] = jnp.dot(u, w2_ref[...], preferred_element_type=F32) * dis[:, None]

    return pl.pallas_call(
        body,
        grid=(n_pad // blk,),
        in_specs=[
            pl.BlockSpec((NC, blk, h), lambda i: (0, i, 0)),
            pl.BlockSpec((blk, h), lambda i: (i, 0)),
            pl.BlockSpec((NC, blk, LANES), lambda i: (0, i, 0)),
            pl.BlockSpec((h, c_pad), lambda i: (0, 0)),
            pl.BlockSpec((1, h), lambda i: (0, 0)),
        ],
        out_specs=pl.BlockSpec((blk, c_pad), lambda i: (i, 0)),
        out_shape=jax.ShapeDtypeStruct((n_pad, c_pad), F32),
    )


def _tc3_call(n, c_pad, c, blk):
    def body(acc_ref, hs2_ref, deg_ref, b2_ref, o_ref):
        dis = _dis_from(deg_ref)
        v = (acc_ref[0] + acc_ref[1] + hs2_ref[...]) * dis[:, None] + b2_ref[...]
        v = v[:, :c]
        m = jnp.max(v, axis=1, keepdims=True)
        lse = jnp.log(jnp.sum(jnp.exp(v - m), axis=1, keepdims=True)) + m
        o_ref[...] = v - lse

    return pl.pallas_call(
        body,
        grid=(n // blk,),
        in_specs=[
            pl.BlockSpec((NC, blk, c_pad), lambda i: (0, i, 0)),
            pl.BlockSpec((blk, c_pad), lambda i: (i, 0)),
            pl.BlockSpec((NC, blk, LANES), lambda i: (0, i, 0)),
            pl.BlockSpec((1, c_pad), lambda i: (0, 0)),
        ],
        out_specs=pl.BlockSpec((blk, c), lambda i: (i, 0)),
        out_shape=jax.ShapeDtypeStruct((n, c), F32),
    )


def kernel(x, edge_index, W1, b1, W2, b2):
    n, d_in = x.shape
    h = W1.shape[1]
    c = W2.shape[1]
    e = edge_index.shape[1]
    blk = 256
    n_pad = -(-(n + 1) // 2048) * 2048
    ch = -(-e // (NW * CHUNK))
    ch = -(-ch // NBUF) * NBUF
    e_pad = NW * ch * CHUNK
    c_pad = -(-c // LANES) * LANES

    src = edge_index[0].astype(jnp.int32)
    dst = edge_index[1].astype(jnp.int32)
    pad_e = e_pad - e
    src3 = jnp.concatenate(
        [src, jnp.zeros((pad_e,), jnp.int32)]).reshape(NW, ch, CHUNK)
    dst3 = jnp.concatenate(
        [dst, jnp.full((pad_e,), n, jnp.int32)]).reshape(NW, ch, CHUNK)

    w2p = jnp.pad(W2, ((0, 0), (0, c_pad - c)))
    b1r = b1.reshape(1, h)
    b2r = jnp.pad(b2, (0, c_pad - c)).reshape(1, c_pad)
    ones16 = jnp.ones((CHUNK, LANES), F32)
    rps = n_pad // NS
    z16 = jnp.zeros((rps, LANES), F32)
    zh = jnp.zeros((rps, h), F32)
    zc = jnp.zeros((rps, c_pad), F32)

    x_p = jnp.pad(x, ((0, n_pad - n), (0, 0)))
    degp = _deg_call(n_pad, ch)(dst3, ones16, z16)
    g1 = _matmul_call(n_pad, d_in, h, blk)(x_p, W1)
    hs1 = _scale_call(n_pad, h, blk)(g1, degp)
    acc1 = _agg_call(n_pad, ch, h)(hs1, src3, dst3, zh)
    hs2 = _tc2_call(n_pad, h, c_pad, blk)(acc1, hs1, degp, w2p, b1r)
    acc2 = _agg_call(n_pad, ch, c_pad)(hs2, src3, dst3, zc)
    tblk = 2000 if n % 2000 == 0 else 400 if n % 400 == 0 else 8
    return _tc3_call(n, c_pad, c, tblk)(acc2, hs2, degp, b2r)


# speedup vs baseline: 1.1086x; 1.1086x over previous
import functools

import jax
import jax.numpy as jnp
from jax import lax
from jax.experimental import pallas as pl
from jax.experimental.pallas import tpu as pltpu
from jax.experimental.pallas import tpu_sc as plsc

F32 = jnp.float32
NC, NS, LANES = 2, 16, 16
NW = NC * NS
CHUNK = 128
NBUF = 4
LAG = 1
_SC_PARAMS = pltpu.CompilerParams(use_tc_tiling_on_sc=False)


def _deg_call(n_pad, ch):
    mesh = plsc.VectorSubcoreMesh(core_axis_name="c", subcore_axis_name="s")
    rows_per_sub = n_pad // NS

    @functools.partial(
        pl.kernel, mesh=mesh, compiler_params=_SC_PARAMS,
        out_type=jax.ShapeDtypeStruct((NC, n_pad, LANES), F32),
        scratch_types=[
            pltpu.VMEM((ch, CHUNK), jnp.int32),
            pltpu.VMEM((CHUNK, LANES), F32),
            pltpu.VMEM_SHARED((n_pad, LANES), F32),
            pltpu.SemaphoreType.DMA,
        ],
    )
    def deg_kernel(dst_hbm, ones_hbm, zeros_hbm, out_hbm, dst_v, ones_v, acc_sh,
                   sem):
        c = lax.axis_index("c")
        s = lax.axis_index("s")
        wid = c * NS + s
        sl = pl.ds(s * rows_per_sub, rows_per_sub)
        pltpu.sync_copy(zeros_hbm, acc_sh.at[sl])
        pltpu.sync_copy(ones_hbm, ones_v)
        pltpu.sync_copy(dst_hbm.at[wid], dst_v)
        plsc.subcore_barrier()

        @pl.loop(0, ch)
        def _(j):
            pltpu.async_copy(ones_v, acc_sh.at[dst_v.at[j]], sem, add=True)

        @pl.loop(0, ch)
        def _(j):
            pltpu.make_async_copy(ones_v, acc_sh.at[dst_v.at[j]], sem).wait()

        plsc.subcore_barrier()
        pltpu.sync_copy(acc_sh.at[sl], out_hbm.at[c, sl])

    return deg_kernel


def _agg_call(n_pad, ch, d):
    mesh = plsc.VectorSubcoreMesh(core_axis_name="c", subcore_axis_name="s")
    rows_per_sub = n_pad // NS

    @functools.partial(
        pl.kernel, mesh=mesh, compiler_params=_SC_PARAMS,
        out_type=jax.ShapeDtypeStruct((NC, n_pad, 128), F32),
        scratch_types=[
            pltpu.VMEM((ch, CHUNK), jnp.int32),
            pltpu.VMEM((ch, CHUNK), jnp.int32),
            pltpu.VMEM((NBUF, CHUNK, d), F32),
            pltpu.VMEM_SHARED((n_pad, d), F32),
            pltpu.VMEM_SHARED((n_pad, d), F32),
        ] + [pltpu.SemaphoreType.DMA] * (2 * NBUF),
    )
    def agg_kernel(hs_hbm, src_hbm, dst_hbm, zeros_hbm, out_hbm,
                   src_v, dst_v, rows_v, hs_sh, acc_sh, *sems):
        gsems, ssems = sems[:NBUF], sems[NBUF:]
        c = lax.axis_index("c")
        s = lax.axis_index("s")
        wid = c * NS + s
        sl = pl.ds(s * rows_per_sub, rows_per_sub)
        pltpu.sync_copy(zeros_hbm, acc_sh.at[sl])
        pltpu.sync_copy(hs_hbm.at[sl], hs_sh.at[sl])
        pltpu.sync_copy(src_hbm.at[wid], src_v)
        pltpu.sync_copy(dst_hbm.at[wid], dst_v)
        plsc.subcore_barrier()

        def gather(j, b):
            pltpu.async_copy(hs_sh.at[src_v.at[j]], rows_v.at[b], gsems[b])

        def wait_gather(j, b):
            pltpu.make_async_copy(hs_sh.at[src_v.at[j]], rows_v.at[b],
                                  gsems[b]).wait()

        def scatter(j, b):
            pltpu.async_copy(rows_v.at[b], acc_sh.at[dst_v.at[j]], ssems[b],
                             add=True)

        def wait_scatter(j, b):
            pltpu.make_async_copy(rows_v.at[b], acc_sh.at[dst_v.at[j]],
                                  ssems[b]).wait()

        for b in range(NBUF):
            gather(b, b)

        @pl.loop(0, ch // NBUF)
        def _(p):
            j0 = p * NBUF
            for b in range(NBUF):
                j = j0 + b
                wait_gather(j, b)
                scatter(j, b)
                bm = (b - LAG) % NBUF

                @pl.when((j >= LAG) & (j + NBUF - LAG < ch))
                def _():
                    wait_scatter(j - LAG, bm)
                    gather(j + NBUF - LAG, bm)

        for k in range(NBUF):
            jj = ch - NBUF + k
            wait_scatter(jj, jj % NBUF)

        plsc.subcore_barrier()
        pltpu.sync_copy(acc_sh.at[sl], out_hbm.at[c, sl, pl.ds(0, d)])

    return agg_kernel


def _dis_from(deg_ref):
    deg = deg_ref[0, :, 0] + deg_ref[1, :, 0]
    return lax.rsqrt(jnp.maximum(deg, 1.0))


def _matmul_call(n_pad, d_in, h, blk):
    def body(x_ref, w_ref, o_ref):
        o_ref[...] = jnp.dot(x_ref[...], w_ref[...], preferred_element_type=F32)

    mblk = 2048
    return pl.pallas_call(
        body,
        grid=(n_pad // mblk,),
        in_specs=[
            pl.BlockSpec((mblk, d_in), lambda i: (i, 0)),
            pl.BlockSpec((d_in, h), lambda i: (0, 0)),
        ],
        out_specs=pl.BlockSpec((mblk, h), lambda i: (i, 0)),
        out_shape=jax.ShapeDtypeStruct((n_pad, h), F32),
    )


def _scale_call(n_pad, h, blk):
    def body(g_ref, deg_ref, o_ref):
        dis = _dis_from(deg_ref)
        o_ref[...] = g_ref[...] * dis[:, None]

    sblk = 2048
    return pl.pallas_call(
        body,
        grid=(n_pad // sblk,),
        in_specs=[
            pl.BlockSpec((sblk, h), lambda i: (i, 0)),
            pl.BlockSpec((NC, sblk, LANES), lambda i: (0, i, 0)),
        ],
        out_specs=pl.BlockSpec((sblk, h), lambda i: (i, 0)),
        out_shape=jax.ShapeDtypeStruct((n_pad, h), F32),
    )


def _tc2_call(n_pad, h, c_pad, blk):
    blk = 2048
    def body(acc_ref, hs1_ref, deg_ref, w2_ref, b1_ref, o_ref):
        dis = _dis_from(deg_ref)
        acc = acc_ref[0, :, :h] + acc_ref[1, :, :h]
        u = (acc + hs1_ref[...]) * dis[:, None] + b1_ref[...]
        u = jnp.maximum(u, 0.0)
        o_ref[...] = jnp.dot(u, w2_ref[...], preferred_element_type=F32) * dis[:, None]

    return pl.pallas_call(
        body,
        grid=(n_pad // blk,),
        in_specs=[
            pl.BlockSpec((NC, blk, 128), lambda i: (0, i, 0)),
            pl.BlockSpec((blk, h), lambda i: (i, 0)),
            pl.BlockSpec((NC, blk, LANES), lambda i: (0, i, 0)),
            pl.BlockSpec((h, c_pad), lambda i: (0, 0)),
            pl.BlockSpec((1, h), lambda i: (0, 0)),
        ],
        out_specs=pl.BlockSpec((blk, c_pad), lambda i: (i, 0)),
        out_shape=jax.ShapeDtypeStruct((n_pad, c_pad), F32),
    )


def _tc3_call(n, c_pad, c, blk):
    def body(acc_ref, hs2_ref, deg_ref, b2_ref, o_ref):
        dis = _dis_from(deg_ref)
        acc = acc_ref[0, :, :c_pad] + acc_ref[1, :, :c_pad]
        v = (acc + hs2_ref[...]) * dis[:, None] + b2_ref[...]
        v = v[:, :c]
        m = jnp.max(v, axis=1, keepdims=True)
        lse = jnp.log(jnp.sum(jnp.exp(v - m), axis=1, keepdims=True)) + m
        o_ref[...] = v - lse

    return pl.pallas_call(
        body,
        grid=(n // blk,),
        in_specs=[
            pl.BlockSpec((NC, blk, 128), lambda i: (0, i, 0)),
            pl.BlockSpec((blk, c_pad), lambda i: (i, 0)),
            pl.BlockSpec((NC, blk, LANES), lambda i: (0, i, 0)),
            pl.BlockSpec((1, c_pad), lambda i: (0, 0)),
        ],
        out_specs=pl.BlockSpec((blk, c), lambda i: (i, 0)),
        out_shape=jax.ShapeDtypeStruct((n, c), F32),
    )


def kernel(x, edge_index, W1, b1, W2, b2):
    n, d_in = x.shape
    h = W1.shape[1]
    c = W2.shape[1]
    e = edge_index.shape[1]
    blk = 256
    n_pad = -(-(n + 1) // 2048) * 2048
    ch = -(-e // (NW * CHUNK))
    ch = -(-ch // NBUF) * NBUF
    e_pad = NW * ch * CHUNK
    c_pad = -(-c // LANES) * LANES

    src = edge_index[0].astype(jnp.int32)
    dst = edge_index[1].astype(jnp.int32)
    pad_e = e_pad - e
    src3 = jnp.concatenate(
        [src, jnp.zeros((pad_e,), jnp.int32)]).reshape(NW, ch, CHUNK)
    dst3 = jnp.concatenate(
        [dst, jnp.full((pad_e,), n, jnp.int32)]).reshape(NW, ch, CHUNK)

    w2p = jnp.pad(W2, ((0, 0), (0, c_pad - c)))
    b1r = b1.reshape(1, h)
    b2r = jnp.pad(b2, (0, c_pad - c)).reshape(1, c_pad)
    ones16 = jnp.ones((CHUNK, LANES), F32)
    rps = n_pad // NS
    z16 = jnp.zeros((rps, LANES), F32)
    zh = jnp.zeros((rps, h), F32)
    zc = jnp.zeros((rps, c_pad), F32)

    x_p = jnp.pad(x, ((0, n_pad - n), (0, 0)))
    degp = _deg_call(n_pad, ch)(dst3, ones16, z16)
    g1 = _matmul_call(n_pad, d_in, h, blk)(x_p, W1)
    hs1 = _scale_call(n_pad, h, blk)(g1, degp)
    acc1 = _agg_call(n_pad, ch, h)(hs1, src3, dst3, zh)
    hs2 = _tc2_call(n_pad, h, c_pad, blk)(acc1, hs1, degp, w2p, b1r)
    acc2 = _agg_call(n_pad, ch, c_pad)(hs2, src3, dst3, zc)
    tblk = 2000 if n % 2000 == 0 else 400 if n % 400 == 0 else 8
    return _tc3_call(n, c_pad, c, tblk)(acc2, hs2, degp, b2r)


# speedup vs baseline: 1.1762x; 1.0610x over previous
import functools

import jax
import jax.numpy as jnp
from jax import lax
from jax.experimental import pallas as pl
from jax.experimental.pallas import tpu as pltpu
from jax.experimental.pallas import tpu_sc as plsc

F32 = jnp.float32
NC, NS, LANES = 2, 16, 16
NW = NC * NS
CHUNK = 128
NBUF = 4
LAG = 1
_SC_PARAMS = pltpu.CompilerParams(use_tc_tiling_on_sc=False)


def _deg_call(n_pad, ch):
    mesh = plsc.VectorSubcoreMesh(core_axis_name="c", subcore_axis_name="s")
    rows_per_sub = n_pad // NS

    @functools.partial(
        pl.kernel, mesh=mesh, compiler_params=_SC_PARAMS,
        out_type=jax.ShapeDtypeStruct((NC, n_pad, LANES), F32),
        scratch_types=[
            pltpu.VMEM((ch, CHUNK), jnp.int32),
            pltpu.VMEM((CHUNK, LANES), F32),
            pltpu.VMEM_SHARED((n_pad, LANES), F32),
            pltpu.SemaphoreType.DMA,
        ],
    )
    def deg_kernel(dst_hbm, ones_hbm, zeros_hbm, out_hbm, dst_v, ones_v, acc_sh,
                   sem):
        c = lax.axis_index("c")
        s = lax.axis_index("s")
        wid = c * NS + s
        sl = pl.ds(s * rows_per_sub, rows_per_sub)
        pltpu.sync_copy(zeros_hbm, acc_sh.at[sl])
        pltpu.sync_copy(ones_hbm, ones_v)
        pltpu.sync_copy(dst_hbm.at[wid], dst_v)
        plsc.subcore_barrier()

        @pl.loop(0, ch)
        def _(j):
            pltpu.async_copy(ones_v, acc_sh.at[dst_v.at[j]], sem, add=True)

        @pl.loop(0, ch)
        def _(j):
            pltpu.make_async_copy(ones_v, acc_sh.at[dst_v.at[j]], sem).wait()

        plsc.subcore_barrier()
        pltpu.sync_copy(acc_sh.at[sl], out_hbm.at[c, sl])

    return deg_kernel


def _agg_call(n_pad, ch, d):
    mesh = plsc.VectorSubcoreMesh(core_axis_name="c", subcore_axis_name="s")
    rows_per_sub = n_pad // NS

    @functools.partial(
        pl.kernel, mesh=mesh, compiler_params=_SC_PARAMS,
        out_type=jax.ShapeDtypeStruct((NC, n_pad, 128), F32),
        scratch_types=[
            pltpu.VMEM((ch, CHUNK), jnp.int32),
            pltpu.VMEM((ch, CHUNK), jnp.int32),
            pltpu.VMEM((NBUF, CHUNK, d), F32),
            pltpu.VMEM_SHARED((n_pad, d), F32),
            pltpu.VMEM_SHARED((n_pad, d), F32),
        ] + [pltpu.SemaphoreType.DMA] * (2 * NBUF),
    )
    def agg_kernel(hs_hbm, src_hbm, dst_hbm, zeros_hbm, out_hbm,
                   src_v, dst_v, rows_v, hs_sh, acc_sh, *sems):
        gsems, ssems = sems[:NBUF], sems[NBUF:]
        c = lax.axis_index("c")
        s = lax.axis_index("s")
        wid = c * NS + s
        sl = pl.ds(s * rows_per_sub, rows_per_sub)
        pltpu.sync_copy(zeros_hbm, acc_sh.at[sl])
        pltpu.sync_copy(hs_hbm.at[sl, pl.ds(0, d)], hs_sh.at[sl])
        pltpu.sync_copy(src_hbm.at[wid], src_v)
        pltpu.sync_copy(dst_hbm.at[wid], dst_v)
        plsc.subcore_barrier()

        def gather(j, b):
            pltpu.async_copy(hs_sh.at[src_v.at[j]], rows_v.at[b], gsems[b])

        def wait_gather(j, b):
            pltpu.make_async_copy(hs_sh.at[src_v.at[j]], rows_v.at[b],
                                  gsems[b]).wait()

        def scatter(j, b):
            pltpu.async_copy(rows_v.at[b], acc_sh.at[dst_v.at[j]], ssems[b],
                             add=True)

        def wait_scatter(j, b):
            pltpu.make_async_copy(rows_v.at[b], acc_sh.at[dst_v.at[j]],
                                  ssems[b]).wait()

        for b in range(NBUF):
            gather(b, b)

        @pl.loop(0, ch // NBUF)
        def _(p):
            j0 = p * NBUF
            for b in range(NBUF):
                j = j0 + b
                wait_gather(j, b)
                scatter(j, b)
                bm = (b - LAG) % NBUF

                @pl.when((j >= LAG) & (j + NBUF - LAG < ch))
                def _():
                    wait_scatter(j - LAG, bm)
                    gather(j + NBUF - LAG, bm)

        for k in range(NBUF):
            jj = ch - NBUF + k
            wait_scatter(jj, jj % NBUF)

        plsc.subcore_barrier()
        pltpu.sync_copy(acc_sh.at[sl], out_hbm.at[c, sl, pl.ds(0, d)])

    return agg_kernel


def _dis_from(deg_ref):
    deg = deg_ref[0, :, 0] + deg_ref[1, :, 0]
    return lax.rsqrt(jnp.maximum(deg, 1.0))


def _matmul_call(n_pad, d_in, h, blk):
    def body(x_ref, w_ref, o_ref):
        o_ref[...] = jnp.dot(x_ref[...], w_ref[...], preferred_element_type=F32)

    mblk = 2048
    return pl.pallas_call(
        body,
        grid=(n_pad // mblk,),
        in_specs=[
            pl.BlockSpec((mblk, d_in), lambda i: (i, 0)),
            pl.BlockSpec((d_in, h), lambda i: (0, 0)),
        ],
        out_specs=pl.BlockSpec((mblk, h), lambda i: (i, 0)),
        out_shape=jax.ShapeDtypeStruct((n_pad, h), F32),
    )


def _scale_call(n_pad, h, blk):
    def body(g_ref, deg_ref, o_ref):
        dis = _dis_from(deg_ref)
        o_ref[:, :h] = g_ref[...] * dis[:, None]

    sblk = 2048
    return pl.pallas_call(
        body,
        grid=(n_pad // sblk,),
        in_specs=[
            pl.BlockSpec((sblk, h), lambda i: (i, 0)),
            pl.BlockSpec((NC, sblk, LANES), lambda i: (0, i, 0)),
        ],
        out_specs=pl.BlockSpec((sblk, 128), lambda i: (i, 0)),
        out_shape=jax.ShapeDtypeStruct((n_pad, 128), F32),
    )


def _tc2_call(n_pad, h, c_pad, blk):
    blk = 2048
    def body(acc_ref, hs1_ref, deg_ref, w2_ref, b1_ref, o_ref):
        dis = _dis_from(deg_ref)
        acc = acc_ref[0, :, :h] + acc_ref[1, :, :h]
        u = (acc + hs1_ref[:, :h]) * dis[:, None] + b1_ref[...]
        u = jnp.maximum(u, 0.0)
        o_ref[:, :c_pad] = (
            jnp.dot(u, w2_ref[...], preferred_element_type=F32) * dis[:, None])

    return pl.pallas_call(
        body,
        grid=(n_pad // blk,),
        in_specs=[
            pl.BlockSpec((NC, blk, 128), lambda i: (0, i, 0)),
            pl.BlockSpec((blk, 128), lambda i: (i, 0)),
            pl.BlockSpec((NC, blk, LANES), lambda i: (0, i, 0)),
            pl.BlockSpec((h, c_pad), lambda i: (0, 0)),
            pl.BlockSpec((1, h), lambda i: (0, 0)),
        ],
        out_specs=pl.BlockSpec((blk, 128), lambda i: (i, 0)),
        out_shape=jax.ShapeDtypeStruct((n_pad, 128), F32),
    )


def _tc3_call(n, c_pad, c, blk):
    def body(acc_ref, hs2_ref, deg_ref, b2_ref, o_ref):
        dis = _dis_from(deg_ref)
        acc = acc_ref[0, :, :c_pad] + acc_ref[1, :, :c_pad]
        v = (acc + hs2_ref[:, :c_pad]) * dis[:, None] + b2_ref[...]
        v = v[:, :c]
        m = jnp.max(v, axis=1, keepdims=True)
        lse = jnp.log(jnp.sum(jnp.exp(v - m), axis=1, keepdims=True)) + m
        o_ref[...] = v - lse

    return pl.pallas_call(
        body,
        grid=(n // blk,),
        in_specs=[
            pl.BlockSpec((NC, blk, 128), lambda i: (0, i, 0)),
            pl.BlockSpec((blk, 128), lambda i: (i, 0)),
            pl.BlockSpec((NC, blk, LANES), lambda i: (0, i, 0)),
            pl.BlockSpec((1, c_pad), lambda i: (0, 0)),
        ],
        out_specs=pl.BlockSpec((blk, c), lambda i: (i, 0)),
        out_shape=jax.ShapeDtypeStruct((n, c), F32),
    )


def kernel(x, edge_index, W1, b1, W2, b2):
    n, d_in = x.shape
    h = W1.shape[1]
    c = W2.shape[1]
    e = edge_index.shape[1]
    blk = 256
    n_pad = -(-(n + 1) // 2048) * 2048
    ch = -(-e // (NW * CHUNK))
    ch = -(-ch // NBUF) * NBUF
    e_pad = NW * ch * CHUNK
    c_pad = -(-c // LANES) * LANES

    src = edge_index[0].astype(jnp.int32)
    dst = edge_index[1].astype(jnp.int32)
    pad_e = e_pad - e
    src3 = jnp.concatenate(
        [src, jnp.zeros((pad_e,), jnp.int32)]).reshape(NW, ch, CHUNK)
    dst3 = jnp.concatenate(
        [dst, jnp.full((pad_e,), n, jnp.int32)]).reshape(NW, ch, CHUNK)

    w2p = jnp.pad(W2, ((0, 0), (0, c_pad - c)))
    b1r = b1.reshape(1, h)
    b2r = jnp.pad(b2, (0, c_pad - c)).reshape(1, c_pad)
    ones16 = jnp.ones((CHUNK, LANES), F32)
    rps = n_pad // NS
    z16 = jnp.zeros((rps, LANES), F32)
    zh = jnp.zeros((rps, h), F32)
    zc = jnp.zeros((rps, c_pad), F32)

    x_p = jnp.pad(x, ((0, n_pad - n), (0, 0)))
    degp = _deg_call(n_pad, ch)(dst3, ones16, z16)
    g1 = _matmul_call(n_pad, d_in, h, blk)(x_p, W1)
    hs1 = _scale_call(n_pad, h, blk)(g1, degp)
    acc1 = _agg_call(n_pad, ch, h)(hs1, src3, dst3, zh)
    hs2 = _tc2_call(n_pad, h, c_pad, blk)(acc1, hs1, degp, w2p, b1r)
    acc2 = _agg_call(n_pad, ch, c_pad)(hs2, src3, dst3, zc)
    tblk = 2000 if n % 2000 == 0 else 400 if n % 400 == 0 else 8
    return _tc3_call(n, c_pad, c, tblk)(acc2, hs2, degp, b2r)
